# baseline (speedup 1.0000x reference)
import jax
import jax.numpy as jnp
from jax.experimental import pallas as pl

L = 512
DMSA = 256
DPAIR = 128
DSTATE = 16
DRBF = 64
L0IN = 32
L0OUT = 16
DEDGE = 32
DHID = 32
TOPK = 128
NTOTALDOFS = 10
TR = 8


def _ln(x, eps=1e-5):
    m = jnp.mean(x, axis=1, keepdims=True)
    v = jnp.mean((x - m) ** 2, axis=1, keepdims=True)
    return (x - m) / jnp.sqrt(v + eps)


def _mm(a, b):
    return jnp.dot(a, b, preferred_element_type=jnp.float32)


def _prep_kernel(msa_ref, state_ref, cas_ref, casT_ref,
                 idx_ref, idxT_ref, rot_ref, rotT_ref,
                 wn_s_ref, wn_t_ref, bn_ref, wff1_ref, bff1_ref,
                 wff2_ref, bff2_ref, wm_s_ref, wm_d_ref,
                 node_ref, sln_ref, mask_ref, dist_ref, nb_ref,
                 relx_ref, rely_ref, relz_ref, nsrc_ref, ndst_ref):
    s = _ln(msa_ref[...])
    sln_ref[...] = s
    st = _ln(state_ref[...])
    node = _mm(s, wn_s_ref[...]) + _mm(st, wn_t_ref[...]) + bn_ref[...]
    node = node + _mm(jax.nn.relu(_mm(node, wff1_ref[...]) + bff1_ref[...]),
                      wff2_ref[...]) + bff2_ref[...]
    node = _ln(node)
    node_ref[...] = node
    nsrc_ref[...] = _mm(node, wm_s_ref[...])
    ndst_ref[...] = _mm(node, wm_d_ref[...])

    rx = casT_ref[0:1, :] - cas_ref[:, 0:1]
    ry = casT_ref[1:2, :] - cas_ref[:, 1:2]
    rz = casT_ref[2:3, :] - cas_ref[:, 2:3]
    relx_ref[...] = rx
    rely_ref[...] = ry
    relz_ref[...] = rz
    dist = jnp.sqrt(rx * rx + ry * ry + rz * rz + 1e-8)
    dist_ref[...] = dist

    sep = idxT_ref[...] - idx_ref[...]
    sm = jnp.maximum(rot_ref[...], rotT_ref[...])
    nb_ref[...] = (jnp.sign(sep) * jnp.log(jnp.abs(sep) + 1.0) * (1.0 / 3.0)
                   * (1.0 - sm))

    row = jax.lax.broadcasted_iota(jnp.int32, (L, L), 0)
    col = jax.lax.broadcasted_iota(jnp.int32, (L, L), 1)
    dg = dist + jnp.where(row == col, 999.9, 0.0)

    bits = jax.lax.bitcast_convert_type(dg, jnp.int32)

    def body(_, carry):
        lo, hi = carry
        mid = lo + (hi - lo) // 2
        cnt = jnp.sum((bits <= mid).astype(jnp.int32), axis=1, keepdims=True)
        ge = cnt >= TOPK
        return jnp.where(ge, lo, mid), jnp.where(ge, mid, hi)

    lo0 = jnp.full((L, 1), -1, jnp.int32)
    hi0 = jnp.full((L, 1), 0x7F7FFFFF, jnp.int32)
    _, thr = jax.lax.fori_loop(0, 32, body, (lo0, hi0))

    less = (bits < thr).astype(jnp.float32)
    eq = (bits == thr).astype(jnp.float32)
    c = jnp.sum(less, axis=1, keepdims=True)
    lt = (row < col).astype(jnp.float32)
    tie_rank = _mm(eq, lt)
    mask = less + eq * (tie_rank < (TOPK - c)).astype(jnp.float32)
    mask_ref[...] = mask


def _edge_msg_kernel(pair_ref, mask_ref, dist_ref, nb_ref,
                     relx_ref, rely_ref, relz_ref, nsrc_ref, ndtile_ref,
                     we_p_ref, we_r_ref, we_n_ref, be_ref,
                     wfe1_ref, bfe1_ref, wfe2_ref, bfe2_ref,
                     wm_e_ref, bmsg_ref, wv_ref,
                     agg_ref, vv_ref):
    pln = _ln(pair_ref[...])

    mu = 2.0 + jax.lax.broadcasted_iota(jnp.int32, (1, DRBF), 1).astype(
        jnp.float32) * (20.0 / (DRBF - 1))
    rbf = jnp.exp(-(((dist_ref[...] - mu) * (DRBF / 20.0)) ** 2))

    edge = (_mm(pln, we_p_ref[...]) + _mm(rbf, we_r_ref[...])
            + nb_ref[...] * we_n_ref[...] + be_ref[...])
    edge = edge + _mm(jax.nn.relu(_mm(edge, wfe1_ref[...]) + bfe1_ref[...]),
                      wfe2_ref[...]) + bfe2_ref[...]
    edge = _ln(edge)

    pre = _mm(edge, wm_e_ref[...]) + ndtile_ref[...] + bmsg_ref[...]
    nsrc = nsrc_ref[...]
    mask = mask_ref[...]

    aggs, vvs = [], []
    zero2 = jnp.zeros((1, 2), jnp.float32)
    for r in range(TR):
        s = slice(r * L, (r + 1) * L)
        m = jax.nn.relu(pre[s] + nsrc[r:r + 1, :]) * mask[s]
        aggs.append(jnp.sum(m, axis=0, keepdims=True))
        w = _mm(m, wv_ref[...])
        rel = jnp.concatenate([relx_ref[s], rely_ref[s], relz_ref[s]], axis=1)
        v0 = jnp.sum(w[:, 0:1] * rel, axis=0, keepdims=True)
        v1 = jnp.sum(w[:, 1:2] * rel, axis=0, keepdims=True)
        vvs.append(jnp.concatenate([v0, v1, zero2], axis=1))
    agg_ref[...] = jnp.concatenate(aggs, axis=0)
    vv_ref[...] = jnp.concatenate(vvs, axis=0)


def _final_kernel(agg_ref, vv_ref, xyz9_ref, rot_ref, sln_ref,
                  ws_ref, bs_ref, wl1_ref,
                  ws0_ref, bs0_ref, wsi_ref, bsi_ref,
                  w1_ref, b1_ref, w2_ref, b2_ref,
                  w3_ref, b3_ref, w4_ref, b4_ref, wout_ref, bout_ref,
                  state_ref, xyzn_ref, alpha_ref):
    h = _mm(agg_ref[...], ws_ref[...]) + bs_ref[...]
    state_ref[...] = h

    xyz9 = xyz9_ref[...]
    rot = rot_ref[...]
    vv = vv_ref[...]

    def xyzf(a, c):
        return rot * xyz9[:, c:c + 1] + (1.0 - rot) * xyz9[:, 3 * a + c:3 * a + c + 1]

    v = [[None] * 3 for _ in range(2)]
    for k in range(2):
        for c in range(3):
            acc = vv[:, 3 * k + c:3 * k + c + 1]
            for a in range(3):
                l1 = xyzf(a, c) - xyzf(1, c)
                acc = acc + l1 * wl1_ref[a:a + 1, k:k + 1]
            v[k][c] = acc

    T = [v[0][c] * 0.1 for c in range(3)]
    R = [v[1][c] * 0.01 for c in range(3)]
    qn = jnp.sqrt(1.0 + R[0] * R[0] + R[1] * R[1] + R[2] * R[2])
    qa = 1.0 / qn
    qb, qc, qd = R[0] / qn, R[1] / qn, R[2] / qn
    aa, bb, cc, dd = qa * qa, qb * qb, qc * qc, qd * qd
    ab, ac, ad = qa * qb, qa * qc, qa * qd
    bc, bd, cd = qb * qc, qb * qd, qc * qd
    rot9 = [[aa + bb - cc - dd, 2 * bc - 2 * ad, 2 * bd + 2 * ac],
            [2 * bc + 2 * ad, aa - bb + cc - dd, 2 * cd - 2 * ab],
            [2 * bd - 2 * ac, 2 * cd + 2 * ab, aa - bb - cc + dd]]
    keep = 1.0 - rot
    for r in range(3):
        for cidx in range(3):
            eye = 1.0 if r == cidx else 0.0
            rot9[r][cidx] = keep * rot9[r][cidx] + rot * eye

    for a in range(3):
        for r in range(3):
            acc = xyz9[:, 3 + r:4 + r] + T[r]
            for j in range(3):
                acc = acc + rot9[r][j] * (xyz9[:, 3 * a + j:3 * a + j + 1]
                                          - xyz9[:, 3 + j:4 + j])
            xyzn_ref[:, 3 * a + r:3 * a + r + 1] = acc

    st = _ln(h)
    si = (_mm(sln_ref[...], ws0_ref[...]) + bs0_ref[...]
          + _mm(st, wsi_ref[...]) + bsi_ref[...])
    si = si + _mm(jax.nn.relu(_mm(jax.nn.relu(si), w1_ref[...]) + b1_ref[...]),
                  w2_ref[...]) + b2_ref[...]
    si = si + _mm(jax.nn.relu(_mm(jax.nn.relu(si), w3_ref[...]) + b3_ref[...]),
                  w4_ref[...]) + b4_ref[...]
    alpha_ref[...] = _mm(jax.nn.relu(si), wout_ref[...]) + bout_ref[...]


def _full(shape):
    return pl.BlockSpec(shape, lambda *_: tuple(0 for _ in shape))


def _f32(shape):
    return jax.ShapeDtypeStruct(shape, jnp.float32)


@jax.jit
def _run(msa, pair, xyz, state, idx, rotation_mask, params):
    p = params
    msa0 = msa[0, 0]
    pairf = pair.reshape(L * L, DPAIR)
    cas = xyz[0, :, 1, :]
    xyz9 = xyz[0].reshape(L, 9)
    idxf = idx[0].astype(jnp.float32).reshape(L, 1)
    rotf = rotation_mask[0].astype(jnp.float32).reshape(L, 1)

    def r1(x):
        return x.reshape(1, -1)

    (node, sln, mask, dist, nb, relx, rely, relz, nsrc, ndst) = pl.pallas_call(
        _prep_kernel,
        out_shape=[_f32((L, L0IN)), _f32((L, DMSA)), _f32((L, L)),
                   _f32((L, L)), _f32((L, L)), _f32((L, L)), _f32((L, L)),
                   _f32((L, L)), _f32((L, DHID)), _f32((L, DHID))],
    )(msa0, state[0], cas, cas.T, idxf, idxf.reshape(1, L),
      rotf, rotf.reshape(1, L),
      p['Wn'][:DMSA], p['Wn'][DMSA:], r1(p['bn']), p['Wff1'], r1(p['bff1']),
      p['Wff2'], r1(p['bff2']),
      p['Wmsg'][:L0IN], p['Wmsg'][L0IN:2 * L0IN])

    def col(x):
        return x.reshape(L * L, 1)

    ndtile = jnp.tile(ndst, (TR, 1))
    blk = TR * L
    colspec = pl.BlockSpec((blk, 1), lambda i: (i, 0))
    agg, vv = pl.pallas_call(
        _edge_msg_kernel,
        grid=(L // TR,),
        in_specs=[
            pl.BlockSpec((blk, DPAIR), lambda i: (i, 0)),
            colspec, colspec, colspec, colspec, colspec, colspec,
            pl.BlockSpec((TR, DHID), lambda i: (i, 0)),
            _full((blk, DHID)),
            _full((DPAIR, DEDGE)), _full((DRBF, DEDGE)), _full((1, DEDGE)),
            _full((1, DEDGE)),
            _full((DEDGE, 2 * DEDGE)), _full((1, 2 * DEDGE)),
            _full((2 * DEDGE, DEDGE)), _full((1, DEDGE)),
            _full((DEDGE, DHID)), _full((1, DHID)), _full((DHID, 2)),
        ],
        out_specs=[pl.BlockSpec((TR, DHID), lambda i: (i, 0)),
                   pl.BlockSpec((TR, 8), lambda i: (i, 0))],
        out_shape=[_f32((L, DHID)), _f32((L, 8))],
    )(pairf, col(mask), col(dist), col(nb), col(relx), col(rely), col(relz),
      nsrc, ndtile,
      p['We'][:DPAIR], p['We'][DPAIR:DPAIR + DRBF], p['We'][DPAIR + DRBF:],
      r1(p['be']), p['Wfe1'], r1(p['bfe1']), p['Wfe2'], r1(p['bfe2']),
      p['Wmsg'][2 * L0IN:], r1(p['bmsg']), p['Wv'])

    state_new, xyzn, alpha = pl.pallas_call(
        _final_kernel,
        out_shape=[_f32((L, L0OUT)), _f32((L, 9)), _f32((L, 2 * NTOTALDOFS))],
    )(agg, vv, xyz9, rotf, sln,
      p['Ws'], r1(p['bs']), p['Wl1'],
      p['Ws0'], r1(p['bs0']), p['Wsi'], r1(p['bsi']),
      p['W1'], r1(p['b1']), p['W2'], r1(p['b2']),
      p['W3'], r1(p['b3']), p['W4'], r1(p['b4']),
      p['Wout'], r1(p['bout']))

    return (xyzn.reshape(1, L, 3, 3), state_new[None],
            alpha.reshape(1, L, NTOTALDOFS, 2))


def kernel(msa, pair, xyz, state, idx, rotation_mask, bond_feats, atom_frames,
           params):
    del bond_feats, atom_frames
    return _run(msa, pair, xyz, state, idx, rotation_mask, params)


# speedup vs baseline: 1.2434x; 1.2434x over previous
import jax
import jax.numpy as jnp
from jax.experimental import pallas as pl

L = 512
DMSA = 256
DPAIR = 128
DSTATE = 16
DRBF = 64
L0IN = 32
L0OUT = 16
DEDGE = 32
DHID = 32
TOPK = 128
NTOTALDOFS = 10
TR = 8


def _ln(x, eps=1e-5):
    m = jnp.mean(x, axis=1, keepdims=True)
    v = jnp.mean((x - m) ** 2, axis=1, keepdims=True)
    return (x - m) / jnp.sqrt(v + eps)


def _mm(a, b):
    return jnp.dot(a, b, preferred_element_type=jnp.float32)


def _prep_kernel(msa_ref, state_ref, cas_ref, casT_ref,
                 idx_ref, idxT_ref, rot_ref, rotT_ref,
                 wn_s_ref, wn_t_ref, bn_ref, wff1_ref, bff1_ref,
                 wff2_ref, bff2_ref, wm_s_ref, wm_d_ref,
                 node_ref, sln_ref, mask_ref, dist_ref, nb_ref,
                 relx_ref, rely_ref, relz_ref, nsrc_ref, ndst_ref):
    s = _ln(msa_ref[...])
    sln_ref[...] = s
    st = _ln(state_ref[...])
    node = _mm(s, wn_s_ref[...]) + _mm(st, wn_t_ref[...]) + bn_ref[...]
    node = node + _mm(jax.nn.relu(_mm(node, wff1_ref[...]) + bff1_ref[...]),
                      wff2_ref[...]) + bff2_ref[...]
    node = _ln(node)
    node_ref[...] = node
    nsrc_ref[...] = _mm(node, wm_s_ref[...])
    ndst_ref[...] = _mm(node, wm_d_ref[...])

    rx = casT_ref[0:1, :] - cas_ref[:, 0:1]
    ry = casT_ref[1:2, :] - cas_ref[:, 1:2]
    rz = casT_ref[2:3, :] - cas_ref[:, 2:3]
    relx_ref[...] = rx
    rely_ref[...] = ry
    relz_ref[...] = rz
    dist = jnp.sqrt(rx * rx + ry * ry + rz * rz + 1e-8)
    dist_ref[...] = dist

    sep = idxT_ref[...] - idx_ref[...]
    sm = jnp.maximum(rot_ref[...], rotT_ref[...])
    nb_ref[...] = (jnp.sign(sep) * jnp.log(jnp.abs(sep) + 1.0) * (1.0 / 3.0)
                   * (1.0 - sm))

    row = jax.lax.broadcasted_iota(jnp.int32, (L, L), 0)
    col = jax.lax.broadcasted_iota(jnp.int32, (L, L), 1)
    dg = dist + jnp.where(row == col, 999.9, 0.0)

    bits = jax.lax.bitcast_convert_type(dg, jnp.int32)

    def body(_, carry):
        lo, hi = carry
        mid = lo + (hi - lo) // 2
        cnt = jnp.sum((bits <= mid).astype(jnp.int32), axis=1, keepdims=True)
        ge = cnt >= TOPK
        return jnp.where(ge, lo, mid), jnp.where(ge, mid, hi)

    lo0 = jnp.full((L, 1), -1, jnp.int32)
    hi0 = jnp.full((L, 1), 0x7F7FFFFF, jnp.int32)
    _, thr = jax.lax.fori_loop(0, 32, body, (lo0, hi0))

    less = (bits < thr).astype(jnp.float32)
    eq = (bits == thr).astype(jnp.float32)
    c = jnp.sum(less, axis=1, keepdims=True)
    lt = (row < col).astype(jnp.float32)
    tie_rank = _mm(eq, lt)
    mask = less + eq * (tie_rank < (TOPK - c)).astype(jnp.float32)
    mask_ref[...] = mask


def _edge_msg_kernel(pair_ref, mask_ref, dist_ref, nb_ref,
                     rel3_ref, nsrc_ref, ndtile_ref,
                     we_p_ref, we_r_ref, we_n_ref, be_ref,
                     wfe1_ref, bfe1_ref, wfe2_ref, bfe2_ref,
                     wm_e_ref, bmsg_ref, wv0_ref, wv1_ref,
                     agg_ref, vv_ref):
    pr = pair_ref[...]
    ones = jnp.full((DPAIR, 1), 1.0 / DPAIR, jnp.float32)
    mean = _mm(pr, ones)
    msq = _mm(pr * pr, ones)
    var = msq - mean * mean
    pln = (pr - mean) * jax.lax.rsqrt(var + 1e-5)

    mu = 2.0 + jax.lax.broadcasted_iota(jnp.int32, (1, DRBF), 1).astype(
        jnp.float32) * (20.0 / (DRBF - 1))
    rbf = jnp.exp(-(((dist_ref[...] - mu) * (DRBF / 20.0)) ** 2))

    edge = (_mm(pln, we_p_ref[...]) + _mm(rbf, we_r_ref[...])
            + nb_ref[...] * we_n_ref[...] + be_ref[...])
    edge = edge + _mm(jax.nn.relu(_mm(edge, wfe1_ref[...]) + bfe1_ref[...]),
                      wfe2_ref[...]) + bfe2_ref[...]
    edge = _ln(edge)

    pre = _mm(edge, wm_e_ref[...]) + ndtile_ref[...] + bmsg_ref[...]

    blk_of = jax.lax.broadcasted_iota(jnp.int32, (TR, TR * L), 1) // L
    rid = jax.lax.broadcasted_iota(jnp.int32, (TR, TR * L), 0)
    sel = (blk_of == rid).astype(jnp.float32)
    blk_ofT = jax.lax.broadcasted_iota(jnp.int32, (TR * L, TR), 0) // L
    ridT = jax.lax.broadcasted_iota(jnp.int32, (TR * L, TR), 1)
    selT = (blk_ofT == ridT).astype(jnp.float32)

    nsrcx = _mm(selT, nsrc_ref[...])
    m = jax.nn.relu(pre + nsrcx) * mask_ref[...]
    agg_ref[...] = _mm(sel, m)

    rel3 = rel3_ref[...]
    w0 = _mm(m, wv0_ref[...])
    w1 = _mm(m, wv1_ref[...])
    vv_ref[:, 0:3] = _mm(sel, w0 * rel3)
    vv_ref[:, 3:6] = _mm(sel, w1 * rel3)
    vv_ref[:, 6:8] = jnp.zeros((TR, 2), jnp.float32)


def _final_kernel(agg_ref, vv_ref, xyz9_ref, rot_ref, sln_ref,
                  ws_ref, bs_ref, wl1_ref,
                  ws0_ref, bs0_ref, wsi_ref, bsi_ref,
                  w1_ref, b1_ref, w2_ref, b2_ref,
                  w3_ref, b3_ref, w4_ref, b4_ref, wout_ref, bout_ref,
                  state_ref, xyzn_ref, alpha_ref):
    h = _mm(agg_ref[...], ws_ref[...]) + bs_ref[...]
    state_ref[...] = h

    xyz9 = xyz9_ref[...]
    rot = rot_ref[...]
    vv = vv_ref[...]

    def xyzf(a, c):
        return rot * xyz9[:, c:c + 1] + (1.0 - rot) * xyz9[:, 3 * a + c:3 * a + c + 1]

    v = [[None] * 3 for _ in range(2)]
    for k in range(2):
        for c in range(3):
            acc = vv[:, 3 * k + c:3 * k + c + 1]
            for a in range(3):
                l1 = xyzf(a, c) - xyzf(1, c)
                acc = acc + l1 * wl1_ref[a:a + 1, k:k + 1]
            v[k][c] = acc

    T = [v[0][c] * 0.1 for c in range(3)]
    R = [v[1][c] * 0.01 for c in range(3)]
    qn = jnp.sqrt(1.0 + R[0] * R[0] + R[1] * R[1] + R[2] * R[2])
    qa = 1.0 / qn
    qb, qc, qd = R[0] / qn, R[1] / qn, R[2] / qn
    aa, bb, cc, dd = qa * qa, qb * qb, qc * qc, qd * qd
    ab, ac, ad = qa * qb, qa * qc, qa * qd
    bc, bd, cd = qb * qc, qb * qd, qc * qd
    rot9 = [[aa + bb - cc - dd, 2 * bc - 2 * ad, 2 * bd + 2 * ac],
            [2 * bc + 2 * ad, aa - bb + cc - dd, 2 * cd - 2 * ab],
            [2 * bd - 2 * ac, 2 * cd + 2 * ab, aa - bb - cc + dd]]
    keep = 1.0 - rot
    for r in range(3):
        for cidx in range(3):
            eye = 1.0 if r == cidx else 0.0
            rot9[r][cidx] = keep * rot9[r][cidx] + rot * eye

    for a in range(3):
        for r in range(3):
            acc = xyz9[:, 3 + r:4 + r] + T[r]
            for j in range(3):
                acc = acc + rot9[r][j] * (xyz9[:, 3 * a + j:3 * a + j + 1]
                                          - xyz9[:, 3 + j:4 + j])
            xyzn_ref[:, 3 * a + r:3 * a + r + 1] = acc

    st = _ln(h)
    si = (_mm(sln_ref[...], ws0_ref[...]) + bs0_ref[...]
          + _mm(st, wsi_ref[...]) + bsi_ref[...])
    si = si + _mm(jax.nn.relu(_mm(jax.nn.relu(si), w1_ref[...]) + b1_ref[...]),
                  w2_ref[...]) + b2_ref[...]
    si = si + _mm(jax.nn.relu(_mm(jax.nn.relu(si), w3_ref[...]) + b3_ref[...]),
                  w4_ref[...]) + b4_ref[...]
    alpha_ref[...] = _mm(jax.nn.relu(si), wout_ref[...]) + bout_ref[...]


def _full(shape):
    return pl.BlockSpec(shape, lambda *_: tuple(0 for _ in shape))


def _f32(shape):
    return jax.ShapeDtypeStruct(shape, jnp.float32)


@jax.jit
def _run(msa, pair, xyz, state, idx, rotation_mask, params):
    p = params
    msa0 = msa[0, 0]
    pairf = pair.reshape(L * L, DPAIR)
    cas = xyz[0, :, 1, :]
    xyz9 = xyz[0].reshape(L, 9)
    idxf = idx[0].astype(jnp.float32).reshape(L, 1)
    rotf = rotation_mask[0].astype(jnp.float32).reshape(L, 1)

    def r1(x):
        return x.reshape(1, -1)

    (node, sln, mask, dist, nb, relx, rely, relz, nsrc, ndst) = pl.pallas_call(
        _prep_kernel,
        out_shape=[_f32((L, L0IN)), _f32((L, DMSA)), _f32((L, L)),
                   _f32((L, L)), _f32((L, L)), _f32((L, L)), _f32((L, L)),
                   _f32((L, L)), _f32((L, DHID)), _f32((L, DHID))],
    )(msa0, state[0], cas, cas.T, idxf, idxf.reshape(1, L),
      rotf, rotf.reshape(1, L),
      p['Wn'][:DMSA], p['Wn'][DMSA:], r1(p['bn']), p['Wff1'], r1(p['bff1']),
      p['Wff2'], r1(p['bff2']),
      p['Wmsg'][:L0IN], p['Wmsg'][L0IN:2 * L0IN])

    def col(x):
        return x.reshape(L * L, 1)

    ndtile = jnp.tile(ndst, (TR, 1))
    rel3 = jnp.stack([relx, rely, relz], axis=-1).reshape(L * L, 3)
    blk = TR * L
    colspec = pl.BlockSpec((blk, 1), lambda i: (i, 0))
    agg, vv = pl.pallas_call(
        _edge_msg_kernel,
        grid=(L // TR,),
        in_specs=[
            pl.BlockSpec((blk, DPAIR), lambda i: (i, 0)),
            colspec, colspec, colspec,
            pl.BlockSpec((blk, 3), lambda i: (i, 0)),
            pl.BlockSpec((TR, DHID), lambda i: (i, 0)),
            _full((blk, DHID)),
            _full((DPAIR, DEDGE)), _full((DRBF, DEDGE)), _full((1, DEDGE)),
            _full((1, DEDGE)),
            _full((DEDGE, 2 * DEDGE)), _full((1, 2 * DEDGE)),
            _full((2 * DEDGE, DEDGE)), _full((1, DEDGE)),
            _full((DEDGE, DHID)), _full((1, DHID)),
            _full((DHID, 1)), _full((DHID, 1)),
        ],
        out_specs=[pl.BlockSpec((TR, DHID), lambda i: (i, 0)),
                   pl.BlockSpec((TR, 8), lambda i: (i, 0))],
        out_shape=[_f32((L, DHID)), _f32((L, 8))],
    )(pairf, col(mask), col(dist), col(nb), rel3,
      nsrc, ndtile,
      p['We'][:DPAIR], p['We'][DPAIR:DPAIR + DRBF], p['We'][DPAIR + DRBF:],
      r1(p['be']), p['Wfe1'], r1(p['bfe1']), p['Wfe2'], r1(p['bfe2']),
      p['Wmsg'][2 * L0IN:], r1(p['bmsg']), p['Wv'][:, 0:1], p['Wv'][:, 1:2])

    state_new, xyzn, alpha = pl.pallas_call(
        _final_kernel,
        out_shape=[_f32((L, L0OUT)), _f32((L, 9)), _f32((L, 2 * NTOTALDOFS))],
    )(agg, vv, xyz9, rotf, sln,
      p['Ws'], r1(p['bs']), p['Wl1'],
      p['Ws0'], r1(p['bs0']), p['Wsi'], r1(p['bsi']),
      p['W1'], r1(p['b1']), p['W2'], r1(p['b2']),
      p['W3'], r1(p['b3']), p['W4'], r1(p['b4']),
      p['Wout'], r1(p['bout']))

    return (xyzn.reshape(1, L, 3, 3), state_new[None],
            alpha.reshape(1, L, NTOTALDOFS, 2))


def kernel(msa, pair, xyz, state, idx, rotation_mask, bond_feats, atom_frames,
           params):
    del bond_feats, atom_frames
    return _run(msa, pair, xyz, state, idx, rotation_mask, params)
